# baseline (speedup 1.0000x reference)
import jax
import jax.numpy as jnp
from jax import lax
from jax.experimental import pallas as pl
from jax.experimental.pallas import tpu as pltpu
from jax.experimental.pallas import tpu_sc as plsc

N_NODES = 10000
D = 256
RANK = 16
RPAD = 128
N_LAYERS = 8

NC, NS = 2, 16
NW = NC * NS
NPT = 320
NPAD = NPT * NW
CH = 80
NCHG = 4
GCH = CH * NCHG
REC = 8
DJ = D // 16


def _mesh():
    return plsc.VectorSubcoreMesh(core_axis_name="c", subcore_axis_name="s",
                                  num_cores=NC, num_subcores=NS)


def _wid():
    return lax.axis_index("s") * NC + lax.axis_index("c")


def _sc_layer_body(x_hbm, src_hbm, w_hbm, dl_hbm, offs_hbm, out_hbm,
                   idxg_v, wg_v, dlg_v, rows_v, offs_v, agg_v,
                   semg0, semg1, semr0, semr1):
    semg = (semg0, semg1)
    semr = (semr0, semr1)
    wid = _wid()
    pltpu.sync_copy(offs_hbm, offs_v)
    opair = offs_v[pl.ds(wid, 16)]
    e0 = opair[0]
    e1 = opair[1]
    a0 = (e0 // 8) * 8
    nch = (e1 - a0 + (CH - 1)) // CH
    ngrp = (nch + (NCHG - 1)) // NCHG

    def copy_group(g, b):
        s = a0 + g * GCH
        pltpu.async_copy(src_hbm.at[pl.ds(s, GCH)],
                         idxg_v.at[pl.ds(b * GCH, GCH)], semg[b])
        pltpu.async_copy(w_hbm.at[pl.ds(s, GCH)],
                         wg_v.at[pl.ds(b * GCH, GCH)], semg[b])
        pltpu.async_copy(dl_hbm.at[pl.ds(s, GCH)],
                         dlg_v.at[pl.ds(b * GCH, GCH)], semg[b])

    def wait_group(b):
        pltpu.make_async_copy(src_hbm.at[pl.ds(0, GCH)],
                              idxg_v.at[pl.ds(b * GCH, GCH)],
                              semg[b]).wait()
        pltpu.make_async_copy(w_hbm.at[pl.ds(0, GCH)],
                              wg_v.at[pl.ds(b * GCH, GCH)], semg[b]).wait()
        pltpu.make_async_copy(dl_hbm.at[pl.ds(0, GCH)],
                              dlg_v.at[pl.ds(b * GCH, GCH)], semg[b]).wait()

    def issue_rows(gb, k, rb):
        pltpu.async_copy(
            x_hbm.at[idxg_v.at[pl.ds(gb * GCH + k * CH, CH)]],
            rows_v.at[pl.ds(rb * CH, CH)], semr[rb])

    def wait_rows(rb):
        pltpu.make_async_copy(x_hbm.at[idxg_v.at[pl.ds(0, CH)]],
                              rows_v.at[pl.ds(rb * CH, CH)],
                              semr[rb]).wait()

    def process(c, gb, k, rb):
        s = a0 + c * CH
        lo = jnp.maximum(e0 - s, 0)
        hi = jnp.maximum(jnp.minimum(e1 - s, CH), lo)

        @plsc.parallel_loop(lo, hi, unroll=4)
        def _edge(i):
            m = gb * GCH + k * CH + i
            w = wg_v[pl.ds(m, 16)][0]
            d = dlg_v[pl.ds(m, 16)][0]
            for j in range(DJ):
                sl = pl.ds(j * 16, 16)
                plsc.addupdate(agg_v.at[d, sl], w * rows_v[rb * CH + i, sl])

    @pl.when(ngrp > 0)
    def _():
        copy_group(0, 0)

    @pl.loop(0, NPT)
    def _zero(r):
        for j in range(DJ):
            agg_v[r, pl.ds(j * 16, 16)] = jnp.zeros((16,), jnp.float32)

    @pl.when(nch > 0)
    def _():
        wait_group(0)
        issue_rows(0, 0, 0)

    nq = (ngrp + 1) // 2

    @pl.loop(0, nq)
    def _q(q):
        for gb in range(2):
            g = 2 * q + gb

            @pl.when(g < ngrp)
            def _g():
                @pl.when(g + 1 < ngrp)
                def _():
                    copy_group(g + 1, 1 - gb)

                for k in range(NCHG):
                    c = g * NCHG + k
                    rb = k % 2

                    @pl.when(c < nch)
                    def _c():
                        wait_rows(rb)

                        @pl.when(c + 1 < nch)
                        def _():
                            if k < NCHG - 1:
                                issue_rows(gb, k + 1, 1 - rb)
                            else:
                                wait_group(1 - gb)
                                issue_rows(1 - gb, 0, 1 - rb)

                        process(c, gb, k, rb)

    pltpu.sync_copy(agg_v, out_hbm.at[pl.ds(wid * NPT, NPT)])


_SC_LAYER = pl.kernel(
    _sc_layer_body,
    out_type=jax.ShapeDtypeStruct((NPAD, D), jnp.float32),
    mesh=_mesh(),
    scratch_types=[
        pltpu.VMEM((2 * GCH,), jnp.int32),
        pltpu.VMEM((2 * GCH + 16,), jnp.float32),
        pltpu.VMEM((2 * GCH + 16,), jnp.int32),
        pltpu.VMEM((2 * CH, D), jnp.float32),
        pltpu.VMEM((48,), jnp.int32),
        pltpu.VMEM((NPT, D), jnp.float32),
        pltpu.SemaphoreType.DMA,
        pltpu.SemaphoreType.DMA,
        pltpu.SemaphoreType.DMA,
        pltpu.SemaphoreType.DMA,
    ],
)


def _sc_gather_body(x_hbm, pa_hbm, nidx_hbm, rows_out, arows_out,
                    idx_v, rows_v, arows_v, sem):
    wid = _wid()
    bpw = nidx_hbm.shape[0] // NW
    base = wid * bpw
    pltpu.sync_copy(nidx_hbm.at[pl.ds(base, bpw)], idx_v)
    pltpu.async_copy(x_hbm.at[idx_v], rows_v, sem).wait()
    pltpu.async_copy(pa_hbm.at[idx_v], arows_v, sem).wait()
    pltpu.sync_copy(rows_v, rows_out.at[pl.ds(base, bpw)])
    pltpu.sync_copy(arows_v, arows_out.at[pl.ds(base, bpw)])


def _make_sc_gather(b):
    bpw = b // NW
    return pl.kernel(
        _sc_gather_body,
        out_type=(jax.ShapeDtypeStruct((b, D), jnp.float32),
                  jax.ShapeDtypeStruct((b, RPAD), jnp.float32)),
        mesh=_mesh(),
        scratch_types=[
            pltpu.VMEM((bpw,), jnp.int32),
            pltpu.VMEM((bpw, D), jnp.float32),
            pltpu.VMEM((bpw, RPAD), jnp.float32),
            pltpu.SemaphoreType.DMA,
        ],
    )


def _mm_relu_body(a_ref, w_ref, b_ref, o_ref):
    acc = jnp.dot(a_ref[...], w_ref[...], preferred_element_type=jnp.float32,
                  precision=lax.Precision.HIGHEST)
    o_ref[...] = jnp.maximum(acc + b_ref[...], 0.0)


def _tc_layer(agg, W, b):
    blk = NPAD // 8
    return pl.pallas_call(
        _mm_relu_body,
        grid=(8,),
        in_specs=[pl.BlockSpec((blk, D), lambda i: (i, 0)),
                  pl.BlockSpec((D, D), lambda i: (0, 0)),
                  pl.BlockSpec((1, D), lambda i: (0, 0))],
        out_specs=pl.BlockSpec((blk, D), lambda i: (i, 0)),
        out_shape=jax.ShapeDtypeStruct((NPAD, D), jnp.float32),
    )(agg, W, b[None, :])


def _final_body(rows_ref, arows_ref, pw_ref, pb_ref, pB_ref, base_ref,
                oov_ref, mask_ref, o_ref):
    g = jnp.dot(rows_ref[...], pw_ref[...], preferred_element_type=jnp.float32,
                precision=lax.Precision.HIGHEST) + pb_ref[...]
    g = g + jnp.dot(arows_ref[...], pB_ref[...],
                    preferred_element_type=jnp.float32,
                    precision=lax.Precision.HIGHEST)
    o = base_ref[...] + oov_ref[...]
    o_ref[...] = jnp.where(mask_ref[...] > 0.0, g, o)


def _tc_final(rows, arows, post_W, post_b, pert_B, base, oov, mask):
    b = rows.shape[0]
    return pl.pallas_call(
        _final_body,
        out_shape=jax.ShapeDtypeStruct((b, D), jnp.float32),
    )(rows, arows, post_W, post_b[None, :], pert_B, base, oov, mask)


def kernel(base_embedding, node_idx, in_vocab, edge_index, edge_weight,
           emb, mps_W, mps_b, post_W, post_b, pert_A, pert_B, oov_emb):
    e = edge_index.shape[1]
    src = edge_index[0]
    dst = edge_index[1]

    order = jnp.argsort(dst)
    ds_ = dst[order]
    srcs = src[order]
    ws = edge_weight[order]
    dl = ds_ - (ds_ // NPT) * NPT

    srcs = jnp.concatenate([srcs, jnp.zeros((GCH,), jnp.int32)])
    ws = jnp.concatenate([ws, jnp.zeros((GCH,), jnp.float32)])
    dl = jnp.concatenate([dl, jnp.zeros((GCH,), jnp.int32)])

    bounds = (jnp.arange(33, dtype=jnp.int32) * NPT).astype(jnp.int32)
    offs = jnp.searchsorted(ds_, bounds).astype(jnp.int32)
    offs = jnp.concatenate([offs, jnp.full((15,), e, jnp.int32)])

    x = jnp.pad(emb, ((0, NPAD - N_NODES), (0, 0)))
    for l in range(N_LAYERS):
        agg = _SC_LAYER(x, srcs, ws, dl, offs)
        x = _tc_layer(agg, mps_W[l], mps_b[l])

    pa_pad = jnp.pad(pert_A, ((0, 0), (0, RPAD - RANK)))
    pb_pad = jnp.pad(pert_B, ((0, RPAD - RANK), (0, 0)))
    rows, arows = _make_sc_gather(node_idx.shape[0])(x, pa_pad, node_idx)
    mask = in_vocab.astype(jnp.float32)[:, None]
    return _tc_final(rows, arows, post_W, post_b, pb_pad,
                     base_embedding, oov_emb, mask)


# speedup vs baseline: 1.0313x; 1.0313x over previous
import jax
import jax.numpy as jnp
from jax import lax
from jax.experimental import pallas as pl
from jax.experimental.pallas import tpu as pltpu
from jax.experimental.pallas import tpu_sc as plsc

N_NODES = 10000
D = 256
RANK = 16
RPAD = 128
N_LAYERS = 8

NC, NS = 2, 16
NW = NC * NS
NPT = 320
NPAD = NPT * NW
CH = 80
NCHG = 4
GCH = CH * NCHG
REC = 8
DJ = D // 16


def _mesh():
    return plsc.VectorSubcoreMesh(core_axis_name="c", subcore_axis_name="s",
                                  num_cores=NC, num_subcores=NS)


def _wid():
    return lax.axis_index("s") * NC + lax.axis_index("c")


def _sc_layer_body(x_hbm, src_hbm, w_hbm, dl_hbm, offs_hbm, out_hbm,
                   idxg_v, wg_v, dlg_v, rows_v, offs_v, agg_v,
                   semg0, semg1, semr0, semr1):
    semg = (semg0, semg1)
    semr = (semr0, semr1)
    wid = _wid()
    pltpu.sync_copy(offs_hbm, offs_v)
    opair = offs_v[pl.ds(wid, 16)]
    e0 = opair[0]
    e1 = opair[1]
    a0 = (e0 // 8) * 8
    nch = (e1 - a0 + (CH - 1)) // CH
    ngrp = (nch + (NCHG - 1)) // NCHG

    def copy_group(g, b):
        s = a0 + g * GCH
        pltpu.async_copy(src_hbm.at[pl.ds(s, GCH)],
                         idxg_v.at[pl.ds(b * GCH, GCH)], semg[b])
        pltpu.async_copy(w_hbm.at[pl.ds(s, GCH)],
                         wg_v.at[pl.ds(b * GCH, GCH)], semg[b])
        pltpu.async_copy(dl_hbm.at[pl.ds(s, GCH)],
                         dlg_v.at[pl.ds(b * GCH, GCH)], semg[b])

    def wait_group(b):
        pltpu.make_async_copy(src_hbm.at[pl.ds(0, GCH)],
                              idxg_v.at[pl.ds(b * GCH, GCH)],
                              semg[b]).wait()
        pltpu.make_async_copy(w_hbm.at[pl.ds(0, GCH)],
                              wg_v.at[pl.ds(b * GCH, GCH)], semg[b]).wait()
        pltpu.make_async_copy(dl_hbm.at[pl.ds(0, GCH)],
                              dlg_v.at[pl.ds(b * GCH, GCH)], semg[b]).wait()

    def issue_rows(gb, k, rb):
        pltpu.async_copy(
            x_hbm.at[idxg_v.at[pl.ds(gb * GCH + k * CH, CH)]],
            rows_v.at[pl.ds(rb * CH, CH)], semr[rb])

    def wait_rows(rb):
        pltpu.make_async_copy(x_hbm.at[idxg_v.at[pl.ds(0, CH)]],
                              rows_v.at[pl.ds(rb * CH, CH)],
                              semr[rb]).wait()

    def process(c, gb, k, rb):
        s = a0 + c * CH
        lo = jnp.maximum(e0 - s, 0)
        hi = jnp.maximum(jnp.minimum(e1 - s, CH), lo)

        @plsc.parallel_loop(lo, hi, unroll=2)
        def _edge(i):
            m = gb * GCH + k * CH + i
            w = wg_v[pl.ds(m, 16)][0]
            d = dlg_v[pl.ds(m, 16)][0]
            for j in range(DJ):
                sl = pl.ds(j * 16, 16)
                plsc.addupdate(agg_v.at[d, sl], w * rows_v[rb * CH + i, sl])

    @pl.when(ngrp > 0)
    def _():
        copy_group(0, 0)

    @pl.loop(0, NPT)
    def _zero(r):
        for j in range(DJ):
            agg_v[r, pl.ds(j * 16, 16)] = jnp.zeros((16,), jnp.float32)

    @pl.when(nch > 0)
    def _():
        wait_group(0)
        issue_rows(0, 0, 0)

    nq = (ngrp + 1) // 2

    @pl.loop(0, nq)
    def _q(q):
        for gb in range(2):
            g = 2 * q + gb

            @pl.when(g < ngrp)
            def _g():
                @pl.when(g + 1 < ngrp)
                def _():
                    copy_group(g + 1, 1 - gb)

                for k in range(NCHG):
                    c = g * NCHG + k
                    rb = k % 2

                    @pl.when(c < nch)
                    def _c():
                        wait_rows(rb)

                        @pl.when(c + 1 < nch)
                        def _():
                            if k < NCHG - 1:
                                issue_rows(gb, k + 1, 1 - rb)
                            else:
                                wait_group(1 - gb)
                                issue_rows(1 - gb, 0, 1 - rb)

                        process(c, gb, k, rb)

    pltpu.sync_copy(agg_v, out_hbm.at[pl.ds(wid * NPT, NPT)])


_SC_LAYER = pl.kernel(
    _sc_layer_body,
    out_type=jax.ShapeDtypeStruct((NPAD, D), jnp.float32),
    mesh=_mesh(),
    scratch_types=[
        pltpu.VMEM((2 * GCH,), jnp.int32),
        pltpu.VMEM((2 * GCH + 16,), jnp.float32),
        pltpu.VMEM((2 * GCH + 16,), jnp.int32),
        pltpu.VMEM((2 * CH, D), jnp.float32),
        pltpu.VMEM((48,), jnp.int32),
        pltpu.VMEM((NPT, D), jnp.float32),
        pltpu.SemaphoreType.DMA,
        pltpu.SemaphoreType.DMA,
        pltpu.SemaphoreType.DMA,
        pltpu.SemaphoreType.DMA,
    ],
)


def _sc_gather_body(x_hbm, pa_hbm, nidx_hbm, rows_out, arows_out,
                    idx_v, rows_v, arows_v, sem):
    wid = _wid()
    bpw = nidx_hbm.shape[0] // NW
    base = wid * bpw
    pltpu.sync_copy(nidx_hbm.at[pl.ds(base, bpw)], idx_v)
    pltpu.async_copy(x_hbm.at[idx_v], rows_v, sem).wait()
    pltpu.async_copy(pa_hbm.at[idx_v], arows_v, sem).wait()
    pltpu.sync_copy(rows_v, rows_out.at[pl.ds(base, bpw)])
    pltpu.sync_copy(arows_v, arows_out.at[pl.ds(base, bpw)])


def _make_sc_gather(b):
    bpw = b // NW
    return pl.kernel(
        _sc_gather_body,
        out_type=(jax.ShapeDtypeStruct((b, D), jnp.float32),
                  jax.ShapeDtypeStruct((b, RPAD), jnp.float32)),
        mesh=_mesh(),
        scratch_types=[
            pltpu.VMEM((bpw,), jnp.int32),
            pltpu.VMEM((bpw, D), jnp.float32),
            pltpu.VMEM((bpw, RPAD), jnp.float32),
            pltpu.SemaphoreType.DMA,
        ],
    )


def _mm_relu_body(a_ref, w_ref, b_ref, o_ref):
    acc = jnp.dot(a_ref[...], w_ref[...], preferred_element_type=jnp.float32,
                  precision=lax.Precision.HIGHEST)
    o_ref[...] = jnp.maximum(acc + b_ref[...], 0.0)


def _tc_layer(agg, W, b):
    blk = NPAD // 8
    return pl.pallas_call(
        _mm_relu_body,
        grid=(8,),
        in_specs=[pl.BlockSpec((blk, D), lambda i: (i, 0)),
                  pl.BlockSpec((D, D), lambda i: (0, 0)),
                  pl.BlockSpec((1, D), lambda i: (0, 0))],
        out_specs=pl.BlockSpec((blk, D), lambda i: (i, 0)),
        out_shape=jax.ShapeDtypeStruct((NPAD, D), jnp.float32),
    )(agg, W, b[None, :])


def _final_body(rows_ref, arows_ref, pw_ref, pb_ref, pB_ref, base_ref,
                oov_ref, mask_ref, o_ref):
    g = jnp.dot(rows_ref[...], pw_ref[...], preferred_element_type=jnp.float32,
                precision=lax.Precision.HIGHEST) + pb_ref[...]
    g = g + jnp.dot(arows_ref[...], pB_ref[...],
                    preferred_element_type=jnp.float32,
                    precision=lax.Precision.HIGHEST)
    o = base_ref[...] + oov_ref[...]
    o_ref[...] = jnp.where(mask_ref[...] > 0.0, g, o)


def _tc_final(rows, arows, post_W, post_b, pert_B, base, oov, mask):
    b = rows.shape[0]
    return pl.pallas_call(
        _final_body,
        out_shape=jax.ShapeDtypeStruct((b, D), jnp.float32),
    )(rows, arows, post_W, post_b[None, :], pert_B, base, oov, mask)


def kernel(base_embedding, node_idx, in_vocab, edge_index, edge_weight,
           emb, mps_W, mps_b, post_W, post_b, pert_A, pert_B, oov_emb):
    e = edge_index.shape[1]
    src = edge_index[0]
    dst = edge_index[1]

    order = jnp.argsort(dst)
    ds_ = dst[order]
    srcs = src[order]
    ws = edge_weight[order]
    dl = ds_ - (ds_ // NPT) * NPT

    srcs = jnp.concatenate([srcs, jnp.zeros((GCH,), jnp.int32)])
    ws = jnp.concatenate([ws, jnp.zeros((GCH,), jnp.float32)])
    dl = jnp.concatenate([dl, jnp.zeros((GCH,), jnp.int32)])

    bounds = (jnp.arange(33, dtype=jnp.int32) * NPT).astype(jnp.int32)
    offs = jnp.searchsorted(ds_, bounds).astype(jnp.int32)
    offs = jnp.concatenate([offs, jnp.full((15,), e, jnp.int32)])

    x = jnp.pad(emb, ((0, NPAD - N_NODES), (0, 0)))
    for l in range(N_LAYERS):
        agg = _SC_LAYER(x, srcs, ws, dl, offs)
        x = _tc_layer(agg, mps_W[l], mps_b[l])

    pa_pad = jnp.pad(pert_A, ((0, 0), (0, RPAD - RANK)))
    pb_pad = jnp.pad(pert_B, ((0, RPAD - RANK), (0, 0)))
    rows, arows = _make_sc_gather(node_idx.shape[0])(x, pa_pad, node_idx)
    mask = in_vocab.astype(jnp.float32)[:, None]
    return _tc_final(rows, arows, post_W, post_b, pb_pad,
                     base_embedding, oov_emb, mask)
